# baseline (device time: 73879 ns/iter reference)
import jax
import jax.numpy as jnp
from jax import lax
from jax.experimental import pallas as pl
from jax.experimental.pallas import tpu as pltpu

N_DEV = 4
B = 64
D = 1024
BG = N_DEV * B
N_PHASE = 6
N_SEM = 3 * N_PHASE


def kernel(x, Win0, Wout0, Win1, Wout1, Win2, Wout2):
    def body(x_ref, win0, wout0, win1, wout1, win2, wout2, out_ref,
             xfull, part, rbuf, send_sems, recv_sems):
        my = lax.axis_index("i")
        my_rows = pl.ds(my * B, B)

        barrier = pltpu.get_barrier_semaphore()
        for d in (1, 2, 3):
            pl.semaphore_signal(barrier, inc=1, device_id=(my ^ d,),
                                device_id_type=pl.DeviceIdType.MESH)
        pl.semaphore_wait(barrier, 3)

        phase_ctr = [0]

        def one_shot(srcs, dsts):
            ph = phase_ctr[0]
            phase_ctr[0] += 1
            rdmas = []
            for d in (1, 2, 3):
                i = 3 * ph + (d - 1)
                rdma = pltpu.make_async_remote_copy(
                    src_ref=srcs(d), dst_ref=dsts(d),
                    send_sem=send_sems.at[i], recv_sem=recv_sems.at[i],
                    device_id=(my ^ d,), device_id_type=pl.DeviceIdType.MESH,
                )
                rdma.start()
                rdmas.append(rdma)
            for r in rdmas:
                r.wait()

        def allgather():
            one_shot(lambda d: xfull.at[my_rows, :],
                     lambda d: xfull.at[my_rows, :])

        def reduce_scatter():
            one_shot(lambda d: part.at[pl.ds((my ^ d) * B, B), :],
                     lambda d: rbuf.at[d - 1])

        xfull[my_rows, :] = x_ref[:, :]
        allgather()

        layers = ((win0, wout0), (win1, wout1), (win2, wout2))
        for k, (win, wout) in enumerate(layers):
            h = jnp.maximum(
                jnp.dot(xfull[:, :], win[:, :],
                        preferred_element_type=jnp.float32), 0.0)
            part[:, :] = jnp.dot(h, wout[:, :],
                                 preferred_element_type=jnp.float32)
            reduce_scatter()
            red = (part[my_rows, :] + rbuf[0, :, :]
                   + rbuf[1, :, :] + rbuf[2, :, :])
            if k < len(layers) - 1:
                xfull[my_rows, :] = red
                allgather()
            else:
                out_ref[:, :] = red

    return pl.pallas_call(
        body,
        out_shape=jax.ShapeDtypeStruct((B, D), jnp.float32),
        in_specs=[pl.BlockSpec(memory_space=pltpu.VMEM)] * 7,
        out_specs=pl.BlockSpec(memory_space=pltpu.VMEM),
        scratch_shapes=[
            pltpu.VMEM((BG, D), jnp.float32),
            pltpu.VMEM((BG, D), jnp.float32),
            pltpu.VMEM((3, B, D), jnp.float32),
            pltpu.SemaphoreType.DMA((N_SEM,)),
            pltpu.SemaphoreType.DMA((N_SEM,)),
        ],
        compiler_params=pltpu.CompilerParams(
            collective_id=0,
            vmem_limit_bytes=100 * 1024 * 1024,
        ),
    )(x, Win0, Wout0, Win1, Wout1, Win2, Wout2)
